# baseline (speedup 1.0000x reference)
import functools

import jax
import jax.numpy as jnp
from jax import lax
from jax.experimental import pallas as pl
from jax.experimental.pallas import tpu as pltpu
from jax.experimental.pallas import tpu_sc as plsc

VOCAB = 1000
VMAIN = 896
VTAILP = 128
B, T = 1024, 50
TPAD = 56
NC, NS = 2, 16
NW = NC * NS
B_PER_W = B // NW
NBUF = 2

_mesh = plsc.VectorSubcoreMesh(
    core_axis_name="c", subcore_axis_name="s", num_cores=NC, num_subcores=NS
)


@functools.partial(
    pl.kernel,
    out_type=jax.ShapeDtypeStruct((B, T, VOCAB), jnp.float32),
    mesh=_mesh,
    scratch_types=[
        pltpu.VMEM((NBUF, 1, TPAD), jnp.int32),
        pltpu.VMEM((NBUF, TPAD, VMAIN), jnp.float32),
        pltpu.VMEM((NBUF, TPAD, VTAILP), jnp.float32),
        pltpu.SemaphoreType.DMA((NBUF,)),
        pltpu.SemaphoreType.DMA((NBUF,)),
        pltpu.SemaphoreType.DMA((NBUF,)),
        pltpu.SemaphoreType.DMA((NBUF,)),
    ],
)
def _gather_kernel(idx_hbm, tmain_hbm, ttail_hbm, out_hbm, idx_v, bufm, buft,
                   isem, gmsem, gtsem, ssem):
    wid = lax.axis_index("s") * NC + lax.axis_index("c")
    base = wid * B_PER_W

    def start_idx(j, nb):
        pltpu.async_copy(idx_hbm.at[base + j], idx_v.at[nb], isem.at[nb])

    def wait_idx(j, nb):
        pltpu.make_async_copy(
            idx_hbm.at[base + j], idx_v.at[nb], isem.at[nb]
        ).wait()

    def start_gathers(j, nb):
        pltpu.async_copy(
            tmain_hbm.at[idx_v.at[nb, 0]], bufm.at[nb], gmsem.at[nb]
        )
        pltpu.async_copy(
            ttail_hbm.at[idx_v.at[nb, 0]], buft.at[nb], gtsem.at[nb]
        )

    def wait_gathers(j, nb):
        pltpu.make_async_copy(
            tmain_hbm.at[idx_v.at[nb, 0]], bufm.at[nb], gmsem.at[nb]
        ).wait()
        pltpu.make_async_copy(
            ttail_hbm.at[idx_v.at[nb, 0]], buft.at[nb], gtsem.at[nb]
        ).wait()

    def scatter_parts(j, nb):
        t48 = pl.multiple_of(48 + 0 * j, 8)
        c896 = pl.multiple_of(VMAIN + 0 * j, 128)
        b = base + j
        return (
            (bufm.at[nb, pl.ds(0, 48)],
             out_hbm.at[b, pl.ds(0, 48), pl.ds(0, VMAIN)]),
            (bufm.at[nb, pl.ds(48, 8)],
             out_hbm.at[b, pl.ds(t48, 8), pl.ds(0, VMAIN)]),
            (buft.at[nb, pl.ds(0, 48)],
             out_hbm.at[b, pl.ds(0, 48), pl.ds(c896, VTAILP)]),
            (buft.at[nb, pl.ds(48, 8)],
             out_hbm.at[b, pl.ds(t48, 8), pl.ds(c896, VTAILP)]),
        )

    def start_scatters(j, nb):
        for src, dst in scatter_parts(j, nb):
            pltpu.async_copy(src, dst, ssem.at[nb])

    def wait_scatters(j, nb):
        for src, dst in scatter_parts(j, nb):
            pltpu.make_async_copy(src, dst, ssem.at[nb]).wait()

    start_idx(0, 0)
    start_idx(1, 1)
    wait_idx(0, 0)
    start_gathers(0, 0)

    wait_gathers(0, 0)
    start_scatters(0, 0)
    start_idx(2, 0)
    wait_idx(1, 1)
    start_gathers(1, 1)

    def body(j, carry):
        nb = lax.rem(j, 2)
        pb = lax.rem(j + 1, 2)
        wait_gathers(j, nb)
        start_scatters(j, nb)
        start_idx(j + 2, nb)
        wait_scatters(j - 1, pb)
        wait_idx(j + 1, pb)
        start_gathers(j + 1, pb)
        return carry

    lax.fori_loop(1, B_PER_W - 2, body, 0)

    j = B_PER_W - 2
    nb, pb = j % 2, (j + 1) % 2
    wait_gathers(j, nb)
    start_scatters(j, nb)
    wait_scatters(j - 1, pb)
    wait_idx(j + 1, pb)
    start_gathers(j + 1, pb)

    j = B_PER_W - 1
    nb, pb = j % 2, (j + 1) % 2
    wait_gathers(j, nb)
    start_scatters(j, nb)
    wait_scatters(j - 1, pb)
    wait_scatters(j, nb)


def kernel(idx, table):
    tmain = table[:, :VMAIN]
    ttail = jnp.pad(table[:, VMAIN:], ((0, 0), (0, VTAILP - (VOCAB - VMAIN))))
    idx_p = jnp.pad(idx, ((0, 0), (0, TPAD - T))).reshape(B, 1, TPAD)
    return _gather_kernel(idx_p, tmain, ttail)
